# baseline (speedup 1.0000x reference)
import functools

import jax
import jax.numpy as jnp
from jax import lax
from jax.experimental import pallas as pl
from jax.experimental.pallas import tpu as pltpu
from jax.experimental.pallas import tpu_sc as plsc

NUM_CORES = 2
NUM_SUBCORES = 16
NUM_WORKERS = NUM_CORES * NUM_SUBCORES
CHUNK = 128


@functools.lru_cache(maxsize=None)
def _make_gather(B, D, n_chunks):
    b_per_w = B // NUM_WORKERS
    assert b_per_w == n_chunks * CHUNK
    mesh = plsc.VectorSubcoreMesh(core_axis_name="c", subcore_axis_name="s")

    @functools.partial(
        pl.kernel,
        mesh=mesh,
        out_type=jax.ShapeDtypeStruct((B, D), jnp.float32),
        scratch_types=[
            pltpu.VMEM((n_chunks, CHUNK), jnp.int32),
            pltpu.VMEM((b_per_w, D), jnp.float32),
            pltpu.SemaphoreType.DMA,
        ],
        compiler_params=pltpu.CompilerParams(use_tc_tiling_on_sc=False),
    )
    def gather_kernel(table_hbm, idx_hbm, out_hbm, idx_v, rows_v, sem):
        wid = lax.axis_index("s") * NUM_CORES + lax.axis_index("c")
        base = wid * b_per_w
        pltpu.sync_copy(idx_hbm.at[wid], idx_v)
        copies = [
            pltpu.async_copy(
                table_hbm.at[idx_v.at[j]],
                rows_v.at[pl.ds(j * CHUNK, CHUNK)],
                sem,
            )
            for j in range(n_chunks)
        ]
        for c in copies:
            c.wait()
        pltpu.sync_copy(rows_v, out_hbm.at[pl.ds(base, b_per_w)])

    return gather_kernel


def kernel(x, embed_weight):
    (B,) = x.shape
    V, D = embed_weight.shape
    n_chunks = B // (NUM_WORKERS * CHUNK)
    idx = x.astype(jnp.int32).reshape(NUM_WORKERS, n_chunks, CHUNK)
    out = _make_gather(B, D, n_chunks)(embed_weight, idx)
    return out[None]


# speedup vs baseline: 2.5740x; 2.5740x over previous
import functools

import jax
import jax.numpy as jnp
from jax import lax
from jax.experimental import pallas as pl
from jax.experimental.pallas import tpu as pltpu
from jax.experimental.pallas import tpu_sc as plsc

NUM_CORES = 2
NUM_SUBCORES = 16
NUM_WORKERS = NUM_CORES * NUM_SUBCORES


@functools.lru_cache(maxsize=None)
def _make_gather(B, D, sub):
    b_per_w = B // NUM_WORKERS
    mesh = plsc.VectorSubcoreMesh(core_axis_name="c", subcore_axis_name="s")

    @functools.partial(
        pl.kernel,
        mesh=mesh,
        out_type=jax.ShapeDtypeStruct((B, D), jnp.float32),
        scratch_types=[
            pltpu.VMEM((b_per_w,), jnp.int32),
            pltpu.VMEM((b_per_w, D), jnp.float32),
            pltpu.SemaphoreType.DMA,
        ],
        compiler_params=pltpu.CompilerParams(
            use_tc_tiling_on_sc=True, needs_layout_passes=False
        ),
    )
    def gather_kernel(tab_hbm, idx_hbm, out_hbm, idx_v, stage, sem):
        wid = lax.axis_index("s") * NUM_CORES + lax.axis_index("c")
        base = wid * b_per_w
        pltpu.sync_copy(idx_hbm.at[wid], idx_v)

        def group_body(g, carry):
            vec = idx_v[pl.ds(g * 16, 16)]
            t_vec = lax.shift_right_logical(vec, 3)
            s_vec = jnp.bitwise_and(vec, sub - 1)
            for l in range(16):
                t = t_vec[l]
                s = s_vec[l]
                pltpu.async_copy(tab_hbm.at[t, s], stage.at[g * 16 + l], sem)
            return carry

        lax.fori_loop(0, b_per_w // 16, group_body, 0)
        pltpu.make_async_copy(
            out_hbm.at[pl.ds(base, b_per_w)], stage, sem
        ).wait()
        pltpu.sync_copy(stage, out_hbm.at[pl.ds(base, b_per_w)])

    return gather_kernel


def kernel(x, embed_weight):
    (B,) = x.shape
    V, D = embed_weight.shape
    sub = 8
    tab3 = embed_weight.reshape(V // sub, sub, D)
    idx = x.astype(jnp.int32).reshape(NUM_WORKERS, B // NUM_WORKERS)
    out = _make_gather(B, D, sub)(tab3, idx)
    return out[None]
